# baseline (speedup 1.0000x reference)
import functools

import jax
import jax.numpy as jnp
from jax import lax
from jax.experimental import pallas as pl
from jax.experimental.pallas import tpu as pltpu
from jax.experimental.pallas import tpu_sc as plsc

N = 16384
BP = 1024
TB = 4096
NP = N // BP
NT = N // TB
LOSS_WEIGHT = 1.0

NC = 2
NS = 16
NW = NC * NS
B_PER_W = N // NW
IDX_W = 128
N_STREAM = B_PER_W // IDX_W


def _argmin_kernel(pred_ref, tgt_ref, idx_ref, smin_ref, sidx_ref):
    ip = pl.program_id(0)
    it = pl.program_id(1)

    @pl.when(it == 0)
    def _init():
        smin_ref[...] = jnp.full((BP, 1), jnp.inf, jnp.float32)
        sidx_ref[...] = jnp.zeros((BP, 1), jnp.int32)

    pred_blk = pred_ref[...]
    tgt_blk = tgt_ref[...]

    pc = pred_blk[:, :3]
    tc = tgt_blk[:3, :]
    psq = pc * pc
    pn = psq[:, 0:1] + psq[:, 1:2] + psq[:, 2:3]
    sq = tc * tc
    tn = sq[0:1, :] + sq[1:2, :] + sq[2:3, :]
    dots = lax.dot_general(
        pc, tc, (((1,), (0,)), ((), ())),
        preferred_element_type=jnp.float32)
    s = (pn + tn) - 2.0 * dots

    m = jnp.min(s, axis=1, keepdims=True)
    col = lax.broadcasted_iota(jnp.int32, (1, TB), 1) + it * TB
    i = jnp.min(jnp.where(s == m, col, jnp.int32(0x7FFFFFFF)),
                axis=1, keepdims=True)

    take = m < smin_ref[...]
    smin_ref[...] = jnp.where(take, m, smin_ref[...])
    sidx_ref[...] = jnp.where(take, i, sidx_ref[...])

    @pl.when(it == NT - 1)
    def _finish():
        idx_ref[...] = sidx_ref[...]


def _sc_gather_kernel(idx_hbm, tint_hbm, out_hbm, idx_v, rows_v, sem):
    wid = lax.axis_index("s") * NC + lax.axis_index("c")
    pltpu.sync_copy(idx_hbm.at[wid], idx_v)
    for j in range(N_STREAM):
        pltpu.async_copy(
            tint_hbm.at[idx_v.at[j]],
            rows_v.at[pl.ds(j * IDX_W, IDX_W)], sem).wait()
    pltpu.sync_copy(rows_v, out_hbm.at[pl.ds(wid * B_PER_W, B_PER_W)])


def _reduce_kernel(pred_ref, matched_ref, out_ref):
    diff = pred_ref[:, 3:4] - matched_ref[:, 0:1]
    out_ref[...] = jnp.full(
        (1, 1), jnp.sum(diff * diff) * (LOSS_WEIGHT / N), jnp.float32)


def kernel(pred, target):
    tgt_t = target.T

    idx = pl.pallas_call(
        _argmin_kernel,
        grid=(NP, NT),
        in_specs=[
            pl.BlockSpec((BP, 4), lambda ip, it: (ip, 0)),
            pl.BlockSpec((4, TB), lambda ip, it: (0, it)),
        ],
        out_specs=pl.BlockSpec((BP, 1), lambda ip, it: (ip, 0)),
        out_shape=jax.ShapeDtypeStruct((N, 1), jnp.int32),
        scratch_shapes=[
            pltpu.VMEM((BP, 1), jnp.float32),
            pltpu.VMEM((BP, 1), jnp.int32),
        ],
        compiler_params=pltpu.CompilerParams(
            dimension_semantics=("arbitrary", "arbitrary")),
    )(pred, tgt_t)

    mesh = plsc.VectorSubcoreMesh(core_axis_name="c", subcore_axis_name="s")
    sc_gather = functools.partial(
        pl.kernel, mesh=mesh,
        out_type=jax.ShapeDtypeStruct((N, 8), jnp.float32),
        scratch_types=[
            pltpu.VMEM((N_STREAM, IDX_W), jnp.int32),
            pltpu.VMEM((B_PER_W, 8), jnp.float32),
            pltpu.SemaphoreType.DMA,
        ],
        compiler_params=pltpu.CompilerParams(use_tc_tiling_on_sc=False),
    )(_sc_gather_kernel)

    tint_table = jnp.broadcast_to(target[:, 3:4], (N, 8))
    matched = sc_gather(jnp.reshape(idx, (NW, N_STREAM, IDX_W)), tint_table)

    out = pl.pallas_call(
        _reduce_kernel,
        out_shape=jax.ShapeDtypeStruct((1, 1), jnp.float32),
    )(pred, matched)
    return jnp.reshape(out, ())


# speedup vs baseline: 1.0291x; 1.0291x over previous
import functools

import jax
import jax.numpy as jnp
from jax import lax
from jax.experimental import pallas as pl
from jax.experimental.pallas import tpu as pltpu
from jax.experimental.pallas import tpu_sc as plsc

N = 16384
BP = 1024
TB = 4096
NP = N // BP
NT = N // TB
LOSS_WEIGHT = 1.0

NC = 2
NS = 16
NW = NC * NS
B_PER_W = N // NW
IDX_W = 128
N_STREAM = B_PER_W // IDX_W


def _argmin_kernel(pred_ref, tgt_ref, idx_ref, smin_ref, sidx_ref):
    ip = pl.program_id(0)
    it = pl.program_id(1)

    @pl.when(it == 0)
    def _init():
        smin_ref[...] = jnp.full((BP, 1), jnp.inf, jnp.float32)
        sidx_ref[...] = jnp.zeros((BP, 1), jnp.float32)

    pred_blk = pred_ref[...]
    tgt_blk = tgt_ref[...]

    pc = pred_blk[:, :3]
    tc = tgt_blk[:3, :]
    sq = tc * tc
    tn = sq[0:1, :] + sq[1:2, :] + sq[2:3, :]
    dots = lax.dot_general(
        pc, tc, (((1,), (0,)), ((), ())),
        preferred_element_type=jnp.float32)
    s = tn - 2.0 * dots

    m = jnp.min(s, axis=1, keepdims=True)
    col = (lax.broadcasted_iota(jnp.int32, (1, TB), 1)
           + it * TB).astype(jnp.float32)
    i = jnp.min((s - m) * 1e30 + col, axis=1, keepdims=True)

    take = m < smin_ref[...]
    smin_ref[...] = jnp.where(take, m, smin_ref[...])
    sidx_ref[...] = jnp.where(take, i, sidx_ref[...])

    @pl.when(it == NT - 1)
    def _finish():
        idx_ref[...] = sidx_ref[...].astype(jnp.int32)


def _sc_gather_kernel(idx_hbm, tint_hbm, out_hbm, idx_v, rows_v, sem):
    wid = lax.axis_index("s") * NC + lax.axis_index("c")
    pltpu.sync_copy(idx_hbm.at[wid], idx_v)
    for j in range(N_STREAM):
        pltpu.async_copy(
            tint_hbm.at[idx_v.at[j]],
            rows_v.at[pl.ds(j * IDX_W, IDX_W)], sem).wait()
    pltpu.sync_copy(rows_v, out_hbm.at[pl.ds(wid * B_PER_W, B_PER_W)])


def _reduce_kernel(pred_ref, matched_ref, out_ref):
    diff = pred_ref[:, 3:4] - matched_ref[:, 0:1]
    out_ref[...] = jnp.full(
        (1, 1), jnp.sum(diff * diff) * (LOSS_WEIGHT / N), jnp.float32)


def kernel(pred, target):
    tgt_t = target.T

    idx = pl.pallas_call(
        _argmin_kernel,
        grid=(NP, NT),
        in_specs=[
            pl.BlockSpec((BP, 4), lambda ip, it: (ip, 0)),
            pl.BlockSpec((4, TB), lambda ip, it: (0, it)),
        ],
        out_specs=pl.BlockSpec((BP, 1), lambda ip, it: (ip, 0)),
        out_shape=jax.ShapeDtypeStruct((N, 1), jnp.int32),
        scratch_shapes=[
            pltpu.VMEM((BP, 1), jnp.float32),
            pltpu.VMEM((BP, 1), jnp.float32),
        ],
        compiler_params=pltpu.CompilerParams(
            dimension_semantics=("arbitrary", "arbitrary")),
    )(pred, tgt_t)

    mesh = plsc.VectorSubcoreMesh(core_axis_name="c", subcore_axis_name="s")
    sc_gather = functools.partial(
        pl.kernel, mesh=mesh,
        out_type=jax.ShapeDtypeStruct((N, 8), jnp.float32),
        scratch_types=[
            pltpu.VMEM((N_STREAM, IDX_W), jnp.int32),
            pltpu.VMEM((B_PER_W, 8), jnp.float32),
            pltpu.SemaphoreType.DMA,
        ],
        compiler_params=pltpu.CompilerParams(use_tc_tiling_on_sc=False),
    )(_sc_gather_kernel)

    tint_table = jnp.broadcast_to(target[:, 3:4], (N, 8))
    matched = sc_gather(jnp.reshape(idx, (NW, N_STREAM, IDX_W)), tint_table)

    out = pl.pallas_call(
        _reduce_kernel,
        out_shape=jax.ShapeDtypeStruct((1, 1), jnp.float32),
    )(pred, matched)
    return jnp.reshape(out, ())


# speedup vs baseline: 1.1157x; 1.0842x over previous
import functools

import jax
import jax.numpy as jnp
from jax import lax
from jax.experimental import pallas as pl
from jax.experimental.pallas import tpu as pltpu
from jax.experimental.pallas import tpu_sc as plsc

N = 16384
BP = 1024
TB = 4096
NP = N // BP
NT = N // TB
LOSS_WEIGHT = 1.0

NC = 2
NS = 16
NW = NC * NS
B_PER_W = N // NW
IDX_W = 128
N_STREAM = B_PER_W // IDX_W


def _argmin_kernel(pred_ref, tgt_ref, idx_ref, smin_ref, sidx_ref):
    ip = pl.program_id(0)
    it = pl.program_id(1)

    @pl.when(it == 0)
    def _init():
        smin_ref[...] = jnp.full((BP, 1), jnp.inf, jnp.float32)
        sidx_ref[...] = jnp.zeros((BP, 1), jnp.float32)

    pred_blk = pred_ref[...]
    tgt_blk = tgt_ref[...]

    pc = pred_blk[:, :3] * -2.0
    tc = tgt_blk[:3, :]
    sq = tc * tc
    tn = sq[0:1, :] + sq[1:2, :] + sq[2:3, :]
    dots2 = lax.dot_general(
        pc, tc, (((1,), (0,)), ((), ())),
        preferred_element_type=jnp.float32)
    s = tn + dots2

    m = jnp.min(s, axis=1, keepdims=True)
    col = (lax.broadcasted_iota(jnp.int32, (1, TB), 1)
           + it * TB).astype(jnp.float32)
    i = jnp.min(jnp.where(s == m, col, jnp.float32(3.0e38)),
                axis=1, keepdims=True)

    take = m < smin_ref[...]
    smin_ref[...] = jnp.where(take, m, smin_ref[...])
    sidx_ref[...] = jnp.where(take, i, sidx_ref[...])

    @pl.when(it == NT - 1)
    def _finish():
        idx_ref[...] = sidx_ref[...].astype(jnp.int32)


def _sc_gather_kernel(idx_hbm, tint_hbm, out_hbm, idx_v, rows_v, sem):
    wid = lax.axis_index("s") * NC + lax.axis_index("c")
    pltpu.sync_copy(idx_hbm.at[wid], idx_v)
    for j in range(N_STREAM):
        pltpu.async_copy(
            tint_hbm.at[idx_v.at[j]],
            rows_v.at[pl.ds(j * IDX_W, IDX_W)], sem).wait()
    pltpu.sync_copy(rows_v, out_hbm.at[pl.ds(wid * B_PER_W, B_PER_W)])


def _reduce_kernel(pred_ref, matched_ref, out_ref):
    diff = pred_ref[:, 3:4] - matched_ref[:, 0:1]
    out_ref[...] = jnp.full(
        (1, 1), jnp.sum(diff * diff) * (LOSS_WEIGHT / N), jnp.float32)


def kernel(pred, target):
    tgt_t = target.T

    idx = pl.pallas_call(
        _argmin_kernel,
        grid=(NP, NT),
        in_specs=[
            pl.BlockSpec((BP, 4), lambda ip, it: (ip, 0)),
            pl.BlockSpec((4, TB), lambda ip, it: (0, it)),
        ],
        out_specs=pl.BlockSpec((BP, 1), lambda ip, it: (ip, 0)),
        out_shape=jax.ShapeDtypeStruct((N, 1), jnp.int32),
        scratch_shapes=[
            pltpu.VMEM((BP, 1), jnp.float32),
            pltpu.VMEM((BP, 1), jnp.float32),
        ],
        compiler_params=pltpu.CompilerParams(
            dimension_semantics=("arbitrary", "arbitrary")),
    )(pred, tgt_t)

    mesh = plsc.VectorSubcoreMesh(core_axis_name="c", subcore_axis_name="s")
    sc_gather = functools.partial(
        pl.kernel, mesh=mesh,
        out_type=jax.ShapeDtypeStruct((N, 8), jnp.float32),
        scratch_types=[
            pltpu.VMEM((N_STREAM, IDX_W), jnp.int32),
            pltpu.VMEM((B_PER_W, 8), jnp.float32),
            pltpu.SemaphoreType.DMA,
        ],
        compiler_params=pltpu.CompilerParams(use_tc_tiling_on_sc=False),
    )(_sc_gather_kernel)

    tint_table = jnp.broadcast_to(target[:, 3:4], (N, 8))
    matched = sc_gather(jnp.reshape(idx, (NW, N_STREAM, IDX_W)), tint_table)

    out = pl.pallas_call(
        _reduce_kernel,
        out_shape=jax.ShapeDtypeStruct((1, 1), jnp.float32),
    )(pred, matched)
    return jnp.reshape(out, ())


# speedup vs baseline: 1.1607x; 1.0403x over previous
import functools

import jax
import jax.numpy as jnp
from jax import lax
from jax.experimental import pallas as pl
from jax.experimental.pallas import tpu as pltpu
from jax.experimental.pallas import tpu_sc as plsc

N = 16384
BP = 1024
TB = 4096
NP = N // BP
NT = N // TB
LOSS_WEIGHT = 1.0

NC = 2
NS = 16
NW = NC * NS
B_PER_W = N // NW
IDX_W = 128
N_STREAM = B_PER_W // IDX_W


def _argmin_kernel(pred_ref, tgt_ref, idx_ref, smin_ref, sidx_ref):
    ip = pl.program_id(0)
    it = pl.program_id(1)

    @pl.when(it == 0)
    def _init():
        smin_ref[...] = jnp.full((BP, 1), jnp.inf, jnp.float32)
        sidx_ref[...] = jnp.zeros((BP, 1), jnp.float32)

    pred_blk = pred_ref[...]
    tgt_blk = tgt_ref[...]

    pc = pred_blk[:, :3] * -2.0
    tc = tgt_blk[:3, :]
    sq = tc * tc
    tn = sq[0:1, :] + sq[1:2, :] + sq[2:3, :]
    dots2 = lax.dot_general(
        pc, tc, (((1,), (0,)), ((), ())),
        preferred_element_type=jnp.float32)
    s = tn + dots2

    m = jnp.min(s, axis=1, keepdims=True)
    col = (lax.broadcasted_iota(jnp.int32, (1, TB), 1)
           + it * TB).astype(jnp.float32)
    i = jnp.min(jnp.where(s == m, col, jnp.float32(3.0e38)),
                axis=1, keepdims=True)

    take = m < smin_ref[...]
    smin_ref[...] = jnp.where(take, m, smin_ref[...])
    sidx_ref[...] = jnp.where(take, i, sidx_ref[...])

    @pl.when(it == NT - 1)
    def _finish():
        idx_ref[...] = jnp.reshape(
            sidx_ref[...].astype(jnp.int32), (1, 1, BP))


def _sc_gather_kernel(idx_hbm, tint_hbm, out_hbm, idx_v, rows_v, sem):
    wid = lax.axis_index("s") * NC + lax.axis_index("c")
    pltpu.sync_copy(idx_hbm.at[wid], idx_v)
    for j in range(N_STREAM):
        pltpu.async_copy(
            tint_hbm.at[idx_v.at[j]],
            rows_v.at[pl.ds(j * IDX_W, IDX_W)], sem).wait()
    pltpu.sync_copy(rows_v, out_hbm.at[pl.ds(wid * B_PER_W, B_PER_W)])


def _reduce_kernel(pred_ref, matched_ref, out_ref):
    diff = pred_ref[...] - matched_ref[...]
    out_ref[...] = jnp.full(
        (1, 1), jnp.sum(diff * diff) * (LOSS_WEIGHT / (8 * N)), jnp.float32)


def kernel(pred, target):
    tgt_t = target.T

    idx = pl.pallas_call(
        _argmin_kernel,
        grid=(NP, NT),
        in_specs=[
            pl.BlockSpec((BP, 4), lambda ip, it: (ip, 0)),
            pl.BlockSpec((4, TB), lambda ip, it: (0, it)),
        ],
        out_specs=pl.BlockSpec((1, 1, BP), lambda ip, it: (ip, 0, 0)),
        out_shape=jax.ShapeDtypeStruct((NP, 1, BP), jnp.int32),
        scratch_shapes=[
            pltpu.VMEM((BP, 1), jnp.float32),
            pltpu.VMEM((BP, 1), jnp.float32),
        ],
        compiler_params=pltpu.CompilerParams(
            dimension_semantics=("arbitrary", "arbitrary")),
    )(pred, tgt_t)

    mesh = plsc.VectorSubcoreMesh(core_axis_name="c", subcore_axis_name="s")
    sc_gather = functools.partial(
        pl.kernel, mesh=mesh,
        out_type=jax.ShapeDtypeStruct((N, 8), jnp.float32),
        scratch_types=[
            pltpu.VMEM((N_STREAM, IDX_W), jnp.int32),
            pltpu.VMEM((B_PER_W, 8), jnp.float32),
            pltpu.SemaphoreType.DMA,
        ],
        compiler_params=pltpu.CompilerParams(use_tc_tiling_on_sc=False),
    )(_sc_gather_kernel)

    tint_table = jnp.broadcast_to(target[:, 3:4], (N, 8))
    matched = sc_gather(jnp.reshape(idx, (NW, N_STREAM, IDX_W)), tint_table)

    pred8 = jnp.reshape(jnp.broadcast_to(pred[:, 3:4], (N, 8)),
                        (N * 8 // 128, 128))
    matched8 = jnp.reshape(matched, (N * 8 // 128, 128))
    out = pl.pallas_call(
        _reduce_kernel,
        out_shape=jax.ShapeDtypeStruct((1, 1), jnp.float32),
    )(pred8, matched8)
    return jnp.reshape(out, ())


# speedup vs baseline: 1.1682x; 1.0065x over previous
import functools

import jax
import jax.numpy as jnp
from jax import lax
from jax.experimental import pallas as pl
from jax.experimental.pallas import tpu as pltpu
from jax.experimental.pallas import tpu_sc as plsc

N = 16384
BP = 1024
TB = 8192
NP = N // BP
NT = N // TB
LOSS_WEIGHT = 1.0

NC = 2
NS = 16
NW = NC * NS
B_PER_W = N // NW
IDX_W = 128
N_STREAM = B_PER_W // IDX_W


def _argmin_kernel(pred_ref, tgt_ref, idx_ref, smin_ref, sidx_ref):
    ip = pl.program_id(0)
    it = pl.program_id(1)

    @pl.when(it == 0)
    def _init():
        smin_ref[...] = jnp.full((BP, 1), jnp.inf, jnp.float32)
        sidx_ref[...] = jnp.zeros((BP, 1), jnp.float32)

    pred_blk = pred_ref[...]
    tgt_blk = tgt_ref[...]

    pc = pred_blk[:, :3] * -2.0
    tc = tgt_blk[:3, :]
    sq = tc * tc
    tn = sq[0:1, :] + sq[1:2, :] + sq[2:3, :]
    dots2 = lax.dot_general(
        pc, tc, (((1,), (0,)), ((), ())),
        preferred_element_type=jnp.float32)
    s = tn + dots2

    m = jnp.min(s, axis=1, keepdims=True)
    col = (lax.broadcasted_iota(jnp.int32, (1, TB), 1)
           + it * TB).astype(jnp.float32)
    i = jnp.min(jnp.where(s == m, col, jnp.float32(3.0e38)),
                axis=1, keepdims=True)

    take = m < smin_ref[...]
    smin_ref[...] = jnp.where(take, m, smin_ref[...])
    sidx_ref[...] = jnp.where(take, i, sidx_ref[...])

    @pl.when(it == NT - 1)
    def _finish():
        idx_ref[...] = jnp.reshape(
            sidx_ref[...].astype(jnp.int32), (1, 1, BP))


def _sc_gather_kernel(idx_hbm, tint_hbm, out_hbm, idx_v, rows_v, sem):
    wid = lax.axis_index("s") * NC + lax.axis_index("c")
    pltpu.sync_copy(idx_hbm.at[wid], idx_v)
    for j in range(N_STREAM):
        pltpu.async_copy(
            tint_hbm.at[idx_v.at[j]],
            rows_v.at[pl.ds(j * IDX_W, IDX_W)], sem).wait()
    pltpu.sync_copy(rows_v, out_hbm.at[pl.ds(wid * B_PER_W, B_PER_W)])


def _reduce_kernel(pred_ref, matched_ref, out_ref):
    diff = pred_ref[...] - matched_ref[...]
    out_ref[...] = jnp.full(
        (1, 1), jnp.sum(diff * diff) * (LOSS_WEIGHT / (8 * N)), jnp.float32)


def kernel(pred, target):
    tgt_t = target.T

    idx = pl.pallas_call(
        _argmin_kernel,
        grid=(NP, NT),
        in_specs=[
            pl.BlockSpec((BP, 4), lambda ip, it: (ip, 0)),
            pl.BlockSpec((4, TB), lambda ip, it: (0, it)),
        ],
        out_specs=pl.BlockSpec((1, 1, BP), lambda ip, it: (ip, 0, 0)),
        out_shape=jax.ShapeDtypeStruct((NP, 1, BP), jnp.int32),
        scratch_shapes=[
            pltpu.VMEM((BP, 1), jnp.float32),
            pltpu.VMEM((BP, 1), jnp.float32),
        ],
        compiler_params=pltpu.CompilerParams(
            dimension_semantics=("arbitrary", "arbitrary")),
    )(pred, tgt_t)

    mesh = plsc.VectorSubcoreMesh(core_axis_name="c", subcore_axis_name="s")
    sc_gather = functools.partial(
        pl.kernel, mesh=mesh,
        out_type=jax.ShapeDtypeStruct((N, 8), jnp.float32),
        scratch_types=[
            pltpu.VMEM((N_STREAM, IDX_W), jnp.int32),
            pltpu.VMEM((B_PER_W, 8), jnp.float32),
            pltpu.SemaphoreType.DMA,
        ],
        compiler_params=pltpu.CompilerParams(use_tc_tiling_on_sc=False),
    )(_sc_gather_kernel)

    tint_table = jnp.broadcast_to(target[:, 3:4], (N, 8))
    matched = sc_gather(jnp.reshape(idx, (NW, N_STREAM, IDX_W)), tint_table)

    pred8 = jnp.reshape(jnp.broadcast_to(pred[:, 3:4], (N, 8)),
                        (N * 8 // 128, 128))
    matched8 = jnp.reshape(matched, (N * 8 // 128, 128))
    out = pl.pallas_call(
        _reduce_kernel,
        out_shape=jax.ShapeDtypeStruct((1, 1), jnp.float32),
    )(pred8, matched8)
    return jnp.reshape(out, ())


# speedup vs baseline: 1.2551x; 1.0744x over previous
import functools

import jax
import jax.numpy as jnp
from jax import lax
from jax.experimental import pallas as pl
from jax.experimental.pallas import tpu as pltpu
from jax.experimental.pallas import tpu_sc as plsc

N = 16384
BP = 1024
TB = 4096
NP = N // BP
NT = N // TB
LOSS_WEIGHT = 1.0

NC = 2
NS = 16
NW = NC * NS
B_PER_W = N // NW
IDX_W = 128
N_STREAM = B_PER_W // IDX_W


def _argmin_kernel(pred_ref, tgt_ref, idx_ref, smin_ref, sidx_ref):
    ip = pl.program_id(0)
    it = pl.program_id(1)

    @pl.when(it == 0)
    def _init():
        smin_ref[...] = jnp.full((BP, 1), jnp.inf, jnp.float32)
        sidx_ref[...] = jnp.zeros((BP, 1), jnp.float32)

    pred_blk = pred_ref[...]
    tgt_blk = tgt_ref[...]

    pc = pred_blk[:, :3] * -2.0
    tc = tgt_blk[:3, :]
    sq = tc * tc
    tn = sq[0:1, :] + sq[1:2, :] + sq[2:3, :]
    dots2 = lax.dot_general(
        pc, tc, (((1,), (0,)), ((), ())),
        preferred_element_type=jnp.float32)
    s = tn + dots2

    col = (lax.broadcasted_iota(jnp.int32, (1, TB), 1)
           + it * TB).astype(jnp.float32)
    sv = s
    iv = None
    while sv.shape[1] > 128:
        w = sv.shape[1] // 2
        sl, sr = sv[:, :w], sv[:, w:]
        mask = sr < sl
        if iv is None:
            iv = jnp.where(mask, col[:, w:], col[:, :w])
        else:
            iv = jnp.where(mask, iv[:, w:], iv[:, :w])
        sv = jnp.minimum(sl, sr)
    m = jnp.min(sv, axis=1, keepdims=True)
    i = jnp.min(jnp.where(sv == m, iv, jnp.float32(3.0e38)),
                axis=1, keepdims=True)

    take = m < smin_ref[...]
    smin_ref[...] = jnp.where(take, m, smin_ref[...])
    sidx_ref[...] = jnp.where(take, i, sidx_ref[...])

    @pl.when(it == NT - 1)
    def _finish():
        idx_ref[...] = jnp.reshape(
            sidx_ref[...].astype(jnp.int32), (1, 1, BP))


def _sc_gather_kernel(idx_hbm, tint_hbm, out_hbm, idx_v, rows_v, sem):
    wid = lax.axis_index("s") * NC + lax.axis_index("c")
    pltpu.sync_copy(idx_hbm.at[wid], idx_v)
    for j in range(N_STREAM):
        pltpu.async_copy(
            tint_hbm.at[idx_v.at[j]],
            rows_v.at[pl.ds(j * IDX_W, IDX_W)], sem).wait()
    pltpu.sync_copy(rows_v, out_hbm.at[pl.ds(wid * B_PER_W, B_PER_W)])


def _reduce_kernel(pred_ref, matched_ref, out_ref):
    diff = pred_ref[...] - matched_ref[...]
    out_ref[...] = jnp.full(
        (1, 1), jnp.sum(diff * diff) * (LOSS_WEIGHT / (8 * N)), jnp.float32)


def kernel(pred, target):
    tgt_t = target.T

    idx = pl.pallas_call(
        _argmin_kernel,
        grid=(NP, NT),
        in_specs=[
            pl.BlockSpec((BP, 4), lambda ip, it: (ip, 0)),
            pl.BlockSpec((4, TB), lambda ip, it: (0, it)),
        ],
        out_specs=pl.BlockSpec((1, 1, BP), lambda ip, it: (ip, 0, 0)),
        out_shape=jax.ShapeDtypeStruct((NP, 1, BP), jnp.int32),
        scratch_shapes=[
            pltpu.VMEM((BP, 1), jnp.float32),
            pltpu.VMEM((BP, 1), jnp.float32),
        ],
        compiler_params=pltpu.CompilerParams(
            dimension_semantics=("arbitrary", "arbitrary")),
    )(pred, tgt_t)

    mesh = plsc.VectorSubcoreMesh(core_axis_name="c", subcore_axis_name="s")
    sc_gather = functools.partial(
        pl.kernel, mesh=mesh,
        out_type=jax.ShapeDtypeStruct((N, 8), jnp.float32),
        scratch_types=[
            pltpu.VMEM((N_STREAM, IDX_W), jnp.int32),
            pltpu.VMEM((B_PER_W, 8), jnp.float32),
            pltpu.SemaphoreType.DMA,
        ],
        compiler_params=pltpu.CompilerParams(use_tc_tiling_on_sc=False),
    )(_sc_gather_kernel)

    tint_table = jnp.broadcast_to(target[:, 3:4], (N, 8))
    matched = sc_gather(jnp.reshape(idx, (NW, N_STREAM, IDX_W)), tint_table)

    pred8 = jnp.reshape(jnp.broadcast_to(pred[:, 3:4], (N, 8)),
                        (N * 8 // 128, 128))
    matched8 = jnp.reshape(matched, (N * 8 // 128, 128))
    out = pl.pallas_call(
        _reduce_kernel,
        out_shape=jax.ShapeDtypeStruct((1, 1), jnp.float32),
    )(pred8, matched8)
    return jnp.reshape(out, ())
